# Initial kernel scaffold; baseline (speedup 1.0000x reference)
import jax
import jax.numpy as jnp
from jax.experimental import pallas as pl


def kernel(x, edge_index, edge_weight, W1, b1, W2, b2, W3, b3):
    raise NotImplementedError("write your pallas kernel here")


# speedup vs baseline: 10.4139x; 10.4139x over previous
import functools

import jax
import jax.numpy as jnp
from jax import lax
from jax.experimental import pallas as pl
from jax.experimental.pallas import tpu as pltpu
from jax.experimental.pallas import tpu_sc as plsc

_N = 10000
_E = 320000
_D = 128
_C = 64

_NP = 10112
_NC = 2
_NS = 16
_NW = _NC * _NS
_CHUNK = 128
_TCH = 79
_EP = _NW * _TCH * _CHUNK
_RPT = _NP // _NS


def _sc_mesh():
    return plsc.VectorSubcoreMesh(core_axis_name="c", subcore_axis_name="s",
                                  num_cores=_NC, num_subcores=_NS)


def _make_deg():
    @functools.partial(
        pl.kernel,
        out_type=jax.ShapeDtypeStruct((_NC, _NP, 16), jnp.float32),
        mesh=_sc_mesh(),
        scratch_types=[
            pltpu.VMEM((_TCH, _CHUNK), jnp.int32),
            pltpu.VMEM((_TCH, _CHUNK), jnp.float32),
            pltpu.VMEM((_CHUNK, 16), jnp.float32),
            pltpu.VMEM_SHARED((_NP, 16), jnp.float32),
        ],
    )
    def deg_kernel(col_hbm, ew_hbm, out_hbm, col_v, ew_v, buf, acc):
        cid = lax.axis_index("c")
        sid = lax.axis_index("s")
        wid = sid * _NC + cid
        base = sid * _RPT

        zf = jnp.zeros((16,), jnp.float32)

        def zero_buf(i, carry):
            buf[i, :] = zf
            return carry

        lax.fori_loop(0, _CHUNK, zero_buf, 0)
        for off, sz in ((0, 128), (128, 128), (256, 128), (384, 128), (512, 120)):
            pltpu.sync_copy(buf.at[pl.ds(0, sz)], acc.at[pl.ds(base + off, sz)])
        plsc.subcore_barrier()

        pltpu.sync_copy(col_hbm.at[wid], col_v)
        pltpu.sync_copy(ew_hbm.at[wid], ew_v)

        ones16 = jnp.ones((16,), jnp.float32)

        def chunk(j, carry):
            def build(g, c2):
                ew16 = ew_v[j, pl.ds(g * 16, 16)]
                for l in range(16):
                    buf[g * 16 + l, :] = ones16 * ew16[l]
                return c2

            lax.fori_loop(0, _CHUNK // 16, build, 0)
            pltpu.sync_copy(buf, acc.at[col_v.at[j]], add=True)
            return carry

        lax.fori_loop(0, _TCH, chunk, 0)
        plsc.subcore_barrier()
        pltpu.sync_copy(acc.at[pl.ds(base, _RPT)],
                        out_hbm.at[cid, pl.ds(base, _RPT)])

    return deg_kernel


def _make_agg(d):
    @functools.partial(
        pl.kernel,
        out_type=jax.ShapeDtypeStruct((_NC, _NP, d), jnp.float32),
        mesh=_sc_mesh(),
        scratch_types=[
            pltpu.VMEM((_TCH, _CHUNK), jnp.int32),
            pltpu.VMEM((_TCH, _CHUNK), jnp.int32),
            pltpu.VMEM((_TCH, _CHUNK), jnp.float32),
            pltpu.VMEM((_CHUNK, d), jnp.float32),
            pltpu.VMEM_SHARED((_NP, d), jnp.float32),
        ],
        compiler_params=pltpu.CompilerParams(use_tc_tiling_on_sc=False),
    )
    def agg_kernel(row_hbm, col_hbm, ew_hbm, y_hbm, out_hbm,
                   row_v, col_v, ew_v, rows, acc):
        cid = lax.axis_index("c")
        sid = lax.axis_index("s")
        wid = sid * _NC + cid
        base = sid * _RPT

        zf = jnp.zeros((16,), jnp.float32)

        def zero_rows(i, carry):
            for k in range(d // 16):
                rows[i, pl.ds(k * 16, 16)] = zf
            return carry

        lax.fori_loop(0, _CHUNK, zero_rows, 0)
        for off, sz in ((0, 128), (128, 128), (256, 128), (384, 128), (512, 120)):
            pltpu.sync_copy(rows.at[pl.ds(0, sz)], acc.at[pl.ds(base + off, sz)])
        plsc.subcore_barrier()

        pltpu.sync_copy(row_hbm.at[wid], row_v)
        pltpu.sync_copy(col_hbm.at[wid], col_v)
        pltpu.sync_copy(ew_hbm.at[wid], ew_v)

        def chunk(j, carry):
            pltpu.sync_copy(y_hbm.at[row_v.at[j]], rows)

            def scale(g, c2):
                ew16 = ew_v[j, pl.ds(g * 16, 16)]
                for l in range(16):
                    s = ew16[l]
                    r = g * 16 + l
                    for k in range(d // 16):
                        sl = pl.ds(k * 16, 16)
                        rows[r, sl] = rows[r, sl] * s
                return c2

            lax.fori_loop(0, _CHUNK // 16, scale, 0)
            pltpu.sync_copy(rows, acc.at[col_v.at[j]], add=True)
            return carry

        lax.fori_loop(0, _TCH, chunk, 0)
        plsc.subcore_barrier()
        pltpu.sync_copy(acc.at[pl.ds(base, _RPT)],
                        out_hbm.at[cid, pl.ds(base, _RPT)])

    return agg_kernel


_deg = _make_deg()
_agg128 = _make_agg(_D)
_agg64 = _make_agg(_C)


def _tc_first(degp, x, w):
    def body(deg_ref, x_ref, w_ref, dinv_ref, y_ref):
        deg = deg_ref[0, :, 0:1] + deg_ref[1, :, 0:1] + 1.0
        dinv = lax.rsqrt(deg)
        dinv_ref[...] = dinv
        xt = jnp.dot(x_ref[...], w_ref[...], preferred_element_type=jnp.float32)
        y_ref[...] = xt * dinv

    return pl.pallas_call(
        body,
        out_shape=[jax.ShapeDtypeStruct((_NP, 1), jnp.float32),
                   jax.ShapeDtypeStruct((_NP, w.shape[1]), jnp.float32)],
    )(degp, x, w)


def _tc_mid(p, y, dinv, b, w):
    def body(p_ref, y_ref, dinv_ref, b_ref, w_ref, o_ref):
        s = p_ref[0] + p_ref[1] + y_ref[...]
        pre = s * dinv_ref[...] + b_ref[...]
        h = jnp.maximum(pre, 0.0)
        t = jnp.dot(h, w_ref[...], preferred_element_type=jnp.float32)
        o_ref[...] = t * dinv_ref[...]

    return pl.pallas_call(
        body,
        out_shape=jax.ShapeDtypeStruct((_NP, w.shape[1]), jnp.float32),
    )(p, y, dinv, b, w)


def _tc_final(p, y, dinv, b):
    def body(p_ref, y_ref, dinv_ref, b_ref, o_ref):
        x1 = (p_ref[0] + p_ref[1] + y_ref[...]) * dinv_ref[...] + b_ref[...]
        m = jnp.max(x1, axis=1, keepdims=True)
        lse = jnp.log(jnp.sum(jnp.exp(x1 - m), axis=1, keepdims=True)) + m
        o_ref[...] = x1 - lse

    return pl.pallas_call(
        body,
        out_shape=jax.ShapeDtypeStruct((_NP, y.shape[1]), jnp.float32),
    )(p, y, dinv, b)


def kernel(x, edge_index, edge_weight, W1, b1, W2, b2, W3, b3):
    pad = _EP - _E
    row3 = jnp.pad(edge_index[0], (0, pad)).reshape(_NW, _TCH, _CHUNK)
    col3 = jnp.pad(edge_index[1], (0, pad)).reshape(_NW, _TCH, _CHUNK)
    ew3 = jnp.pad(edge_weight, (0, pad)).reshape(_NW, _TCH, _CHUNK)

    xp = jnp.pad(x, ((0, _NP - _N), (0, 0)))
    degp = _deg(col3, ew3)
    dinv, y1 = _tc_first(degp, xp, W1)
    p1 = _agg128(row3, col3, ew3, y1)
    y2 = _tc_mid(p1, y1, dinv, b1.reshape(1, -1), W2)
    p2 = _agg128(row3, col3, ew3, y2)
    y3 = _tc_mid(p2, y2, dinv, b2.reshape(1, -1), W3)
    p3 = _agg64(row3, col3, ew3, y3)
    return _tc_final(p3, y3, dinv, b3.reshape(1, -1))[:_N]
